# baseline (speedup 1.0000x reference)
import functools

import jax
import jax.numpy as jnp
from jax import lax
from jax.experimental import pallas as pl
from jax.experimental.pallas import tpu as pltpu
from jax.experimental.pallas import tpu_sc as plsc

_D = 32
_B = 16384
_H = 50
_NC = 2
_NS = 16
_NW = _NC * _NS
_B_PER_W = _B // _NW
_HC = 5
_NHC = _H // _HC
_NBB = _B_PER_W // 128
_SZ = _HC * 128
_S_PER_W = _NHC * _NBB
_NB = 2

_mesh = plsc.VectorSubcoreMesh(core_axis_name="c", subcore_axis_name="s")


@functools.partial(
    pl.kernel,
    out_type=jax.ShapeDtypeStruct((_H, _D // 8, _B // 128, 8, 128),
                                  jnp.float32),
    mesh=_mesh,
    compiler_params=pltpu.CompilerParams(use_tc_tiling_on_sc=False,
                                         needs_layout_passes=False),
    scratch_types=(
        [pltpu.VMEM((_SZ,), jnp.int32) for _ in range(_NB)]
        + [
            pltpu.VMEM((_B_PER_W * _H,), jnp.int32),
            pltpu.VMEM((_NB, _SZ, _D), jnp.float32),
            pltpu.VMEM((_NB, _HC, _D // 8, 8, 128), jnp.float32),
            pltpu.SemaphoreType.DMA,
            pltpu.SemaphoreType.DMA((_NB,)),
            pltpu.SemaphoreType.DMA((_NB,)),
        ]
    ),
)
def _gather_kernel(idx_hbm, table_hbm, out_hbm, *scr):
    off_b = scr[:_NB]
    idx_v, rows_v, tv, sem_i, sem_g, sem_o = scr[_NB:]
    wid = lax.axis_index("s") * _NC + lax.axis_index("c")

    pltpu.async_copy(
        idx_hbm.at[pl.ds(wid * _B_PER_W * _H, _B_PER_W * _H)], idx_v,
        sem_i).wait()

    lanes = lax.broadcasted_iota(jnp.int32, (16,), 0)

    def gather_cp(j):
        return pltpu.make_async_copy(
            table_hbm.at[off_b[j]], rows_v.at[j], sem_g.at[j])

    def out_cp(s, j):
        hc = s // _NBB
        bbg = wid * _NBB + lax.rem(s, _NBB)
        return pltpu.make_async_copy(
            tv.at[j],
            out_hbm.at[pl.ds(hc * _HC, _HC), pl.ds(0, _D // 8), bbg],
            sem_o.at[j])

    def build_offsets(s, j):
        hc = s // _NBB
        bb = lax.rem(s, _NBB)
        base = bb * 128 * _H + hc * _HC
        @plsc.parallel_loop(0, _HC * 8, unroll=4)
        def o_body(t):
            ho = t // 8
            bl0 = lax.rem(t, 8) * 16
            src = (bl0 + lanes) * _H + (base + ho)
            off_b[j][pl.ds(ho * 128 + bl0, 16)] = plsc.load_gather(
                idx_v, [src])

    def transpose(j):
        @plsc.parallel_loop(0, _HC * (_D // 8), unroll=8)
        def t_body(t):
            ho = t // (_D // 8)
            fb = lax.rem(t, _D // 8)
            for fs in range(8):
                for bl0 in range(0, 128, 16):
                    rows = ho * 128 + bl0 + lanes
                    cols = jnp.full((16,), fb * 8 + fs, jnp.int32)
                    tv[j, ho, fb, fs, pl.ds(bl0, 16)] = plsc.load_gather(
                        rows_v.at[j], [rows, cols])

    def body(i, carry):
        s0 = _NB * i
        for j in range(_NB):
            @pl.when(i > 0)
            def _(j=j, s=s0 + j):
                out_cp(s - _NB, j).wait()
            build_offsets(s0 + j, j)
            gather_cp(j).start()
        for j in range(_NB):
            gather_cp(j).wait()
            transpose(j)
            out_cp(s0 + j, j).start()
        return carry

    lax.fori_loop(0, _S_PER_W // _NB, body, 0)
    for j in range(_NB):
        out_cp(_S_PER_W - _NB + j, j).wait()


def kernel(x, emb_weight):
    idx = x.reshape(-1).astype(jnp.int32)
    out = _gather_kernel(idx, emb_weight)
    out = out.transpose(2, 4, 0, 1, 3)
    return out.reshape(x.shape + (emb_weight.shape[1],))


# speedup vs baseline: 1.0671x; 1.0671x over previous
import functools

import jax
import jax.numpy as jnp
from jax import lax
from jax.experimental import pallas as pl
from jax.experimental.pallas import tpu as pltpu
from jax.experimental.pallas import tpu_sc as plsc

_D = 32
_B = 16384
_H = 50
_NC = 2
_NS = 16
_NW = _NC * _NS
_B_PER_W = _B // _NW
_HC = 5
_NHC = _H // _HC
_NBB = _B_PER_W // 128
_SZ = _HC * 128
_S_PER_W = _NHC * _NBB
_NB = 2

_mesh = plsc.VectorSubcoreMesh(core_axis_name="c", subcore_axis_name="s")


@functools.partial(
    pl.kernel,
    out_type=jax.ShapeDtypeStruct((_H, _D // 8, _B // 128, 8, 128),
                                  jnp.float32),
    mesh=_mesh,
    compiler_params=pltpu.CompilerParams(use_tc_tiling_on_sc=False,
                                         needs_layout_passes=False),
    scratch_types=(
        [pltpu.VMEM((_SZ,), jnp.int32) for _ in range(_NB)]
        + [
            pltpu.VMEM((_B_PER_W * _H,), jnp.int32),
            pltpu.VMEM((_NB, _SZ, _D), jnp.float32),
            pltpu.VMEM((_NB, _HC, _D // 8, 8, 128), jnp.float32),
            pltpu.SemaphoreType.DMA,
            pltpu.SemaphoreType.DMA((_NB,)),
            pltpu.SemaphoreType.DMA((_NB,)),
        ]
    ),
)
def _gather_kernel(idx_hbm, table_hbm, out_hbm, *scr):
    off_b = scr[:_NB]
    idx_v, rows_v, tv, sem_i, sem_g, sem_o = scr[_NB:]
    wid = lax.axis_index("s") * _NC + lax.axis_index("c")

    pltpu.async_copy(
        idx_hbm.at[pl.ds(wid * _B_PER_W * _H, _B_PER_W * _H)], idx_v,
        sem_i).wait()

    lanes = lax.broadcasted_iota(jnp.int32, (16,), 0)

    def gather_cp(j):
        return pltpu.make_async_copy(
            table_hbm.at[off_b[j]], rows_v.at[j], sem_g.at[j])

    def out_cp(s, j):
        hc = s // _NBB
        bbg = wid * _NBB + lax.rem(s, _NBB)
        return pltpu.make_async_copy(
            tv.at[j],
            out_hbm.at[pl.ds(hc * _HC, _HC), pl.ds(0, _D // 8), bbg],
            sem_o.at[j])

    def build_offsets(s, j):
        hc = s // _NBB
        bb = lax.rem(s, _NBB)
        base = bb * 128 * _H + hc * _HC
        @plsc.parallel_loop(0, _HC * 8, unroll=4)
        def o_body(t):
            ho = t // 8
            bl0 = lax.rem(t, 8) * 16
            src = (bl0 + lanes) * _H + (base + ho)
            off_b[j][pl.ds(ho * 128 + bl0, 16)] = plsc.load_gather(
                idx_v, [src])

    def transpose(j):
        @plsc.parallel_loop(0, _HC * (_D // 8), unroll=4)
        def t_body(t):
            ho = t // (_D // 8)
            fb = lax.rem(t, _D // 8)
            for fs in range(8):
                for bl0 in range(0, 128, 16):
                    rows = ho * 128 + bl0 + lanes
                    cols = jnp.full((16,), fb * 8 + fs, jnp.int32)
                    tv[j, ho, fb, fs, pl.ds(bl0, 16)] = plsc.load_gather(
                        rows_v.at[j], [rows, cols])

    def body(i, carry):
        s0 = _NB * i
        for j in range(_NB):
            @pl.when(i > 0)
            def _(j=j, s=s0 + j):
                out_cp(s - _NB, j).wait()
            build_offsets(s0 + j, j)
            gather_cp(j).start()
        for j in range(_NB):
            gather_cp(j).wait()
            transpose(j)
            out_cp(s0 + j, j).start()
        return carry

    lax.fori_loop(0, _S_PER_W // _NB, body, 0)
    for j in range(_NB):
        out_cp(_S_PER_W - _NB + j, j).wait()


def kernel(x, emb_weight):
    idx = x.reshape(-1).astype(jnp.int32)
    out = _gather_kernel(idx, emb_weight)
    out = out.transpose(2, 4, 0, 1, 3)
    return out.reshape(x.shape + (emb_weight.shape[1],))


# speedup vs baseline: 1.0679x; 1.0007x over previous
import functools

import jax
import jax.numpy as jnp
from jax import lax
from jax.experimental import pallas as pl
from jax.experimental.pallas import tpu as pltpu
from jax.experimental.pallas import tpu_sc as plsc

_D = 32
_B = 16384
_H = 50
_NC = 2
_NS = 16
_NW = _NC * _NS
_B_PER_W = _B // _NW
_HC = 5
_NHC = _H // _HC
_NBB = _B_PER_W // 128
_SZ = _HC * 128
_S_PER_W = _NHC * _NBB
_NB = 2

_mesh = plsc.VectorSubcoreMesh(core_axis_name="c", subcore_axis_name="s")


@functools.partial(
    pl.kernel,
    out_type=jax.ShapeDtypeStruct((_H, _D // 8, _B // 128, 8, 128),
                                  jnp.float32),
    mesh=_mesh,
    compiler_params=pltpu.CompilerParams(use_tc_tiling_on_sc=False,
                                         needs_layout_passes=False),
    scratch_types=(
        [pltpu.VMEM((_SZ,), jnp.int32) for _ in range(_NB)]
        + [
            pltpu.VMEM((_B_PER_W * _H,), jnp.int32),
            pltpu.VMEM((_NB, _SZ, _D), jnp.float32),
            pltpu.VMEM((_NB, _HC, _D // 8, 8, 128), jnp.float32),
            pltpu.SemaphoreType.DMA,
            pltpu.SemaphoreType.DMA((_NB,)),
            pltpu.SemaphoreType.DMA((_NB,)),
        ]
    ),
)
def _gather_kernel(idx_hbm, table_hbm, out_hbm, *scr):
    off_b = scr[:_NB]
    idx_v, rows_2d, tv, sem_i, sem_g, sem_o = scr[_NB:]
    wid = lax.axis_index("s") * _NC + lax.axis_index("c")

    pltpu.async_copy(
        idx_hbm.at[pl.ds(wid * _B_PER_W * _H, _B_PER_W * _H)], idx_v,
        sem_i).wait()

    lanes = lax.broadcasted_iota(jnp.int32, (16,), 0)
    lanes_h = lanes * _H
    lanes_d = lanes * _D

    def gather_cp(j):
        return pltpu.make_async_copy(
            table_hbm.at[off_b[j]], rows_2d.at[j], sem_g.at[j])

    def out_cp(s, j):
        hc = s // _NBB
        bbg = wid * _NBB + lax.rem(s, _NBB)
        return pltpu.make_async_copy(
            tv.at[j],
            out_hbm.at[pl.ds(hc * _HC, _HC), pl.ds(0, _D // 8), bbg],
            sem_o.at[j])

    def build_offsets(s, j):
        hc = s // _NBB
        bb = lax.rem(s, _NBB)
        base = bb * 128 * _H + hc * _HC
        @plsc.parallel_loop(0, _HC * 8, unroll=4)
        def o_body(t):
            ho = t // 8
            bl0 = lax.rem(t, 8) * 16
            src = lanes_h + (bl0 * _H + base + ho)
            off_b[j][pl.ds(ho * 128 + bl0, 16)] = plsc.load_gather(
                idx_v, [src])

    def transpose(j):
        @plsc.parallel_loop(0, _HC * (_D // 8), unroll=4)
        def t_body(t):
            ho = t // (_D // 8)
            fb = lax.rem(t, _D // 8)
            base = ho * 128
            cols = jnp.full((16,), fb * 8, jnp.int32)
            for fs in range(8):
                for bl0 in range(0, 128, 16):
                    rows = lanes + (base + bl0)
                    tv[j, ho, fb, fs, pl.ds(bl0, 16)] = plsc.load_gather(
                        rows_2d.at[j], [rows, cols + fs])

    def body(i, carry):
        s0 = _NB * i
        for j in range(_NB):
            @pl.when(i > 0)
            def _(j=j, s=s0 + j):
                out_cp(s - _NB, j).wait()
            build_offsets(s0 + j, j)
            gather_cp(j).start()
        for j in range(_NB):
            gather_cp(j).wait()
            transpose(j)
            out_cp(s0 + j, j).start()
        return carry

    lax.fori_loop(0, _S_PER_W // _NB, body, 0)
    for j in range(_NB):
        out_cp(_S_PER_W - _NB + j, j).wait()


def kernel(x, emb_weight):
    idx = x.reshape(-1).astype(jnp.int32)
    out = _gather_kernel(idx, emb_weight)
    out = out.transpose(2, 4, 0, 1, 3)
    return out.reshape(x.shape + (emb_weight.shape[1],))
